# baseline (speedup 1.0000x reference)
import functools

import jax
import jax.numpy as jnp
from jax import lax
from jax.experimental import pallas as pl
from jax.experimental.pallas import tpu as pltpu
from jax.experimental.pallas import tpu_sc as plsc

N = 10000
E = 160000
D = 256
L_MP = 4

NW = 32
NB = 12288
EPW = E // NW
EPW_PAD = 5120
CW = 128
NCH = EPW_PAD // CW
SRT_LEN = E + 12288

ROWS = 1000


@functools.lru_cache(maxsize=None)
def _build_sort_kernels():
  smesh = plsc.VectorSubcoreMesh(core_axis_name="c", subcore_axis_name="s")

  @functools.partial(
      pl.kernel,
      out_type=jax.ShapeDtypeStruct((NW, NB), jnp.int32),
      mesh=smesh,
      compiler_params=pltpu.CompilerParams(needs_layout_passes=False),
      scratch_types=[
          pltpu.VMEM((EPW_PAD,), jnp.int32),
          pltpu.VMEM((NB + 16,), jnp.int32),
      ],
  )
  def _sc_hist(dst_hbm, hist_hbm, dslab, hist):
    w = lax.axis_index("c") * 16 + lax.axis_index("s")
    pltpu.sync_copy(dst_hbm.at[pl.ds(w * EPW, EPW_PAD)], dslab)
    zz = jnp.zeros((16,), jnp.int32)

    def zr(i, _):
      hist[pl.ds(i * 16, 16)] = zz
      return 0

    lax.fori_loop(0, (NB + 16) // 16, zr, 0)

    def hr(k, _):
      d16 = dslab[pl.ds(k * 16, 16)]
      gi = k * 16 + lax.iota(jnp.int32, 16)
      d16 = jnp.where(gi < EPW, d16, NB)
      cnt, last = plsc.scan_count(d16)
      plsc.addupdate_scatter(hist, [d16], cnt, mask=last)
      return 0

    lax.fori_loop(0, EPW_PAD // 16, hr, 0)
    pltpu.sync_copy(hist.at[pl.ds(0, NB)], hist_hbm.at[w])

  @functools.partial(
      pl.kernel,
      out_type=(
          jax.ShapeDtypeStruct((NW, NB), jnp.int32),
          jax.ShapeDtypeStruct((NB,), jnp.int32),
      ),
      mesh=smesh,
      compiler_params=pltpu.CompilerParams(needs_layout_passes=False),
      scratch_types=[
          pltpu.VMEM((NW, 384), jnp.int32),
          pltpu.VMEM((NW, 384), jnp.int32),
          pltpu.VMEM((384,), jnp.int32),
      ],
  )
  def _sc_prefix(hist_hbm, wbase_hbm, tot_hbm, hv, wb, tot):
    t = lax.axis_index("c") * 16 + lax.axis_index("s")
    pltpu.sync_copy(hist_hbm.at[:, pl.ds(t * 384, 384)], hv)

    def col(kc, _):
      def row(wr, acc):
        wb[wr, pl.ds(kc * 16, 16)] = acc
        return acc + hv[wr, pl.ds(kc * 16, 16)]

      acc = lax.fori_loop(0, NW, row, jnp.zeros((16,), jnp.int32))
      tot[pl.ds(kc * 16, 16)] = acc
      return 0

    lax.fori_loop(0, 384 // 16, col, 0)
    pltpu.sync_copy(wb, wbase_hbm.at[:, pl.ds(t * 384, 384)])
    pltpu.sync_copy(tot, tot_hbm.at[pl.ds(t * 384, 384)])

  @functools.partial(
      pl.kernel,
      out_type=(
          jax.ShapeDtypeStruct((SRT_LEN,), jnp.int32),
          jax.ShapeDtypeStruct((NB + 16,), jnp.int32),
      ),
      mesh=smesh,
      compiler_params=pltpu.CompilerParams(needs_layout_passes=False),
      scratch_types=[
          pltpu.VMEM((EPW_PAD,), jnp.int32),
          pltpu.VMEM((EPW_PAD,), jnp.int32),
          pltpu.VMEM((NCH, CW), jnp.int32),
          pltpu.VMEM((NB + 16,), jnp.int32),
          pltpu.VMEM((NB + 16,), jnp.int32),
      ],
  )
  def _sc_scatter(dst_hbm, src_hbm, wbase_hbm, tot_hbm, srt_hbm, boff_hbm,
                  dslab, sslab, pos, boff, nxt):
    w = lax.axis_index("c") * 16 + lax.axis_index("s")
    pltpu.sync_copy(dst_hbm.at[pl.ds(w * EPW, EPW_PAD)], dslab)
    pltpu.sync_copy(src_hbm.at[pl.ds(w * EPW, EPW_PAD)], sslab)
    pltpu.sync_copy(tot_hbm, boff.at[pl.ds(0, NB)])
    pltpu.sync_copy(wbase_hbm.at[w], nxt.at[pl.ds(0, NB)])

    def pref(k, carry):
      v = boff[pl.ds(k * 16, 16)]
      cs = plsc.cumsum(v)
      excl = cs - v + carry
      boff[pl.ds(k * 16, 16)] = excl
      nxt[pl.ds(k * 16, 16)] = nxt[pl.ds(k * 16, 16)] + excl
      return carry + jnp.sum(v)

    lax.fori_loop(0, NB // 16, pref, jnp.int32(0))
    boff[pl.ds(NB, 16)] = jnp.zeros((16,), jnp.int32) + E
    nxt[pl.ds(NB, 16)] = jnp.zeros((16,), jnp.int32) + (E + w * CW)

    def sc(k, _):
      d16 = dslab[pl.ds(k * 16, 16)]
      gi = k * 16 + lax.iota(jnp.int32, 16)
      d16 = jnp.where(gi < EPW, d16, NB)
      cnt, last = plsc.scan_count(d16)
      base = plsc.load_gather(nxt, [d16])
      pos[k // 8, pl.ds((k % 8) * 16, 16)] = base + cnt - 1
      plsc.store_scatter(nxt, [d16], base + cnt, mask=last)
      return 0

    lax.fori_loop(0, EPW_PAD // 16, sc, 0)

    def dma(g, _):
      pltpu.sync_copy(sslab.at[pl.ds(g * CW, CW)], srt_hbm.at[pos.at[g]])
      return 0

    lax.fori_loop(0, NCH, dma, 0)

    @pl.when(w == 0)
    def _():
      pltpu.sync_copy(boff, boff_hbm)

  return _sc_hist, _sc_prefix, _sc_scatter


NPR = 320
MAXE_W = 5760
GB = 128
OB = 64
NEG = -3.0e38


@functools.lru_cache(maxsize=None)
def _build_agg_kernel():
  smesh = plsc.VectorSubcoreMesh(core_axis_name="c", subcore_axis_name="s")

  @functools.partial(
      pl.kernel,
      out_type=(
          jax.ShapeDtypeStruct((32 * NPR, 256), jnp.float32),
          jax.ShapeDtypeStruct((32 * NPR, 256), jnp.float32),
      ),
      mesh=smesh,
      compiler_params=pltpu.CompilerParams(needs_layout_passes=False),
      scratch_types=[
          pltpu.VMEM((MAXE_W + 16,), jnp.int32),
          pltpu.VMEM((MAXE_W + 16,), jnp.int32),
          pltpu.VMEM((2, GB, 256), jnp.float32),
          pltpu.VMEM((OB, 256), jnp.float32),
          pltpu.VMEM((OB, 256), jnp.float32),
          pltpu.VMEM((NPR + 16,), jnp.int32),
          pltpu.SemaphoreType.DMA,
          pltpu.SemaphoreType.DMA,
      ],
  )
  def _sc_agg(m2_hbm, srt_hbm, boff_hbm, sum_hbm, max_hbm,
              slab, gidx, stage, outs, outm, offs, sem0, sem1):
    nr = lax.axis_index("c") * 16 + lax.axis_index("s")
    nd0 = nr * NPR
    pltpu.sync_copy(boff_hbm.at[pl.ds(nd0, NPR + 16)], offs)

    lanes = lax.iota(jnp.int32, 16)

    def lane0(v):
      return jnp.sum(jnp.where(lanes == 0, v, 0))

    def lane1(v):
      return jnp.sum(jnp.where(lanes == 1, v, 0))

    def bin_deg(cu):
      v = offs[pl.ds(cu, 16)]
      return lane1(v) - lane0(v)

    e0 = lane0(offs[pl.ds(0, 16)])
    e1 = lane0(offs[pl.ds(NPR, 16)])
    ec = e1 - e0
    e0a = (e0 // 8) * 8
    sh = e0 - e0a
    pltpu.sync_copy(srt_hbm.at[pl.ds(e0a, MAXE_W)], slab.at[pl.ds(0, MAXE_W)])

    def tr(k, _):
      v = slab[pl.ds(sh + k * 16, 16)]
      gi = k * 16 + lax.iota(jnp.int32, 16)
      gidx[pl.ds(k * 16, 16)] = jnp.where(gi < ec, v, 0)
      return 0

    lax.fori_loop(0, (MAXE_W + 16) // 16 - 1, tr, 0)

    nch = (ec + GB - 1) // GB
    sems = (sem0, sem1)

    def issue(g, b):
      pltpu.async_copy(m2_hbm.at[gidx.at[pl.ds(g * GB, GB)]],
                       stage.at[b], sems[b])

    @pl.when(nch > 0)
    def _():
      issue(0, 0)

    @pl.when(nch > 1)
    def _():
      issue(1, 1)

    zero16 = jnp.zeros((16,), jnp.float32)
    neg16 = jnp.zeros((16,), jnp.float32) + NEG

    def flush_body(ob):
      row = pl.multiple_of(nd0 + ob, OB)
      pltpu.sync_copy(outs, sum_hbm.at[pl.ds(row, OB)])
      pltpu.sync_copy(outm, max_hbm.at[pl.ds(row, OB)])

    def chunk_step(g, b, carry):
      def work(carry):
        pltpu.make_async_copy(m2_hbm.at[gidx.at[pl.ds(0, GB)]],
                              stage.at[b], sems[b]).wait()
        jn = jnp.minimum(GB, ec - g * GB)

        def more(st):
          return st[0] < jn

        def per_bin(st):
          j, cur, rem, obase = st[0], st[1], st[2], st[3]
          accs = st[4:20]
          accm = st[20:36]

          def adv_cond(cr):
            return cr[1] == 0

          def adv(cr):
            cu = cr[0] + 1
            return (cu, bin_deg(cu))

          cur, rem = lax.while_loop(adv_cond, adv, (cur, rem))
          take = jnp.minimum(rem, jn - j)

          def e(t, a):
            row = [stage[b, j + t, pl.ds(k * 16, 16)] for k in range(16)]
            return (tuple(a[k] + row[k] for k in range(16))
                    + tuple(jnp.maximum(a[16 + k], row[k]) for k in range(16)))

          aa = lax.fori_loop(0, take, e, tuple(accs) + tuple(accm))
          accs, accm = aa[:16], aa[16:]
          j = j + take
          rem = rem - take

          def fin(args):
            obase, accs, accm = args

            def fl_cond(ob):
              return cur >= ob + OB

            def fl(ob):
              flush_body(ob)
              return ob + OB

            obase = lax.while_loop(fl_cond, fl, obase)
            r = cur - obase
            for k in range(16):
              outs[r, pl.ds(k * 16, 16)] = accs[k]
              outm[r, pl.ds(k * 16, 16)] = accm[k]
            return (obase,
                    tuple(zero16 for _ in range(16)),
                    tuple(neg16 for _ in range(16)))

          def nofin(args):
            obase, accs, accm = args
            return (obase, tuple(accs), tuple(accm))

          obase, accs, accm = lax.cond(
              rem == 0, fin, nofin, (obase, accs, accm))
          return (j, cur, rem, obase) + tuple(accs) + tuple(accm)

        st = (jnp.int32(0),) + tuple(carry)
        st = lax.while_loop(more, per_bin, st)
        carry = st[1:]

        @pl.when(g + 2 < nch)
        def _():
          issue(g + 2, b)

        return carry

      return lax.cond(g < nch, work, lambda c: c, carry)

    init = (jnp.int32(0), bin_deg(jnp.int32(0)), jnp.int32(0))
    init = init + tuple(jnp.zeros((16,), jnp.float32) for _ in range(16))
    init = init + tuple(jnp.zeros((16,), jnp.float32) + NEG for _ in range(16))

    def outer(g2, carry):
      carry = chunk_step(g2 * 2, 0, carry)
      carry = chunk_step(g2 * 2 + 1, 1, carry)
      return tuple(carry)

    carry = lax.fori_loop(0, (nch + 1) // 2, outer, init)
    flush_body(carry[2])

  return _sc_agg


def _sort_edges(edge_index):
    sc_hist, sc_prefix, sc_scatter = _build_sort_kernels()
    pad = jnp.zeros((EPW_PAD,), jnp.int32)
    dst_pad = jnp.concatenate([edge_index[1], pad])
    src_pad = jnp.concatenate([edge_index[0], pad])
    hist = sc_hist(dst_pad)
    wbase, totals = sc_prefix(hist)
    srt, boff = sc_scatter(dst_pad, src_pad, wbase, totals)
    return srt, boff, totals


def _linear_relu_kernel(x_ref, w_ref, b_ref, o_ref):
    o_ref[...] = jax.nn.relu(
        jnp.dot(x_ref[...], w_ref[...], preferred_element_type=jnp.float32)
        + b_ref[...]
    )


def _linear_kernel(x_ref, w_ref, b_ref, o_ref):
    o_ref[...] = (
        jnp.dot(x_ref[...], w_ref[...], preferred_element_type=jnp.float32)
        + b_ref[...]
    )


def _linear(x, w, b, relu=False):
    n = x.shape[0]
    return pl.pallas_call(
        _linear_relu_kernel if relu else _linear_kernel,
        grid=(n // ROWS,),
        in_specs=[
            pl.BlockSpec((ROWS, x.shape[1]), lambda i: (i, 0)),
            pl.BlockSpec((x.shape[1], w.shape[1]), lambda i: (0, 0)),
            pl.BlockSpec((1, w.shape[1]), lambda i: (0, 0)),
        ],
        out_specs=pl.BlockSpec((ROWS, w.shape[1]), lambda i: (i, 0)),
        out_shape=jax.ShapeDtypeStruct((n, w.shape[1]), jnp.float32),
    )(x, w, b.reshape(1, -1))


NBA = 16 * NPR


def _layer_a_kernel(h_ref, wct_ref, bct_ref, wc_ref, bc_ref, m_ref, p_ref):
    h = h_ref[...]
    wct = wct_ref[...]
    bct = bct_ref[...]
    ts = []
    for j in range(5):
        t = jnp.sum(h * wct[j:j + 1, :], axis=1, keepdims=True) + bct[0, j]
        ts.append(jax.nn.relu(t))
    l0, l1, a0, a1, a2 = ts
    mx01 = jnp.maximum(l0, l1)
    e0 = jnp.exp(l0 - mx01)
    e1 = jnp.exp(l1 - mx01)
    d01 = e0 + e1
    a2 = a2 - 1.0
    mx3 = jnp.maximum(jnp.maximum(a0, a1), a2)
    f0 = jnp.exp(a0 - mx3)
    f1 = jnp.exp(a1 - mx3)
    f2 = jnp.exp(a2 - mx3)
    d3 = f0 + f1 + f2
    z = jnp.zeros_like(l0)
    p_ref[...] = jnp.concatenate(
        [f0 / d3, f1 / d3, f2 / d3, e0 / d01, e1 / d01, z, z, z], axis=1)
    m_ref[...] = (
        jnp.dot(h, wc_ref[...], preferred_element_type=jnp.float32)
        + bc_ref[...]
    )


def _layer_a(h, wct, bct, wc, bc):
    return pl.pallas_call(
        _layer_a_kernel,
        grid=(N // ROWS,),
        in_specs=[
            pl.BlockSpec((ROWS, D), lambda i: (i, 0)),
            pl.BlockSpec((8, D), lambda i: (0, 0)),
            pl.BlockSpec((1, 8), lambda i: (0, 0)),
            pl.BlockSpec((D, D), lambda i: (0, 0)),
            pl.BlockSpec((1, D), lambda i: (0, 0)),
        ],
        out_specs=[
            pl.BlockSpec((ROWS, D), lambda i: (i, 0)),
            pl.BlockSpec((ROWS, 8), lambda i: (i, 0)),
        ],
        out_shape=(
            jax.ShapeDtypeStruct((N, D), jnp.float32),
            jax.ShapeDtypeStruct((N, 8), jnp.float32),
        ),
    )(h, wct, bct, wc, bc)


def _layer_b_kernel(s_ref, x_ref, m_ref, p_ref, deg_ref,
                    cp_ref, sel_ref, h2_ref, sel2_ref, cp2_ref):
    s = s_ref[...]
    mx = x_ref[...]
    m = m_ref[...]
    p = p_ref[...]
    degc = deg_ref[...]
    cp = cp_ref[...]
    pos = degc > 0.0
    mean = jnp.where(pos, s / jnp.maximum(degc, 1.0), 0.0)
    mxa = jnp.where(pos, mx, 0.0)
    out = p[:, 0:1] * mean + p[:, 1:2] * mxa + p[:, 2:3] * m
    h2 = jax.nn.relu(out)
    h2_ref[...] = h2
    sel2_ref[...] = sel_ref[...] + (cp * p[:, 4:5]) * h2
    cp2_ref[...] = cp * p[:, 3:4]


def _layer_b(sumS, maxS, m, p, deg_col, cp, sel):
    return pl.pallas_call(
        _layer_b_kernel,
        grid=(N // ROWS,),
        in_specs=[
            pl.BlockSpec((ROWS, D), lambda i: (i, 0)),
            pl.BlockSpec((ROWS, D), lambda i: (i, 0)),
            pl.BlockSpec((ROWS, D), lambda i: (i, 0)),
            pl.BlockSpec((ROWS, 8), lambda i: (i, 0)),
            pl.BlockSpec((ROWS, 1), lambda i: (i, 0)),
            pl.BlockSpec((ROWS, 1), lambda i: (i, 0)),
            pl.BlockSpec((ROWS, D), lambda i: (i, 0)),
        ],
        out_specs=[
            pl.BlockSpec((ROWS, D), lambda i: (i, 0)),
            pl.BlockSpec((ROWS, D), lambda i: (i, 0)),
            pl.BlockSpec((ROWS, 1), lambda i: (i, 0)),
        ],
        out_shape=(
            jax.ShapeDtypeStruct((N, D), jnp.float32),
            jax.ShapeDtypeStruct((N, D), jnp.float32),
            jax.ShapeDtypeStruct((N, 1), jnp.float32),
        ),
    )(sumS[:N], maxS[:N], m, p, deg_col, cp, sel)


def kernel(x, edge_index, params):
    srt, boff, totals = _sort_edges(edge_index)
    sc_agg = _build_agg_kernel()
    deg_col = totals[:N].astype(jnp.float32).reshape(N, 1)

    z3 = jnp.zeros((3, D), dtype=jnp.float32)
    wct = jnp.concatenate([params['W_dc'].T, params['W_ac'].T, z3], axis=0)
    bct = jnp.concatenate(
        [params['b_dc'], params['b_ac'], jnp.zeros((3,), jnp.float32)]
    ).reshape(1, 8)

    h = _linear(x, params['W_pre'], params['b_pre'], relu=True)
    cp = jnp.ones((N, 1), dtype=jnp.float32)
    selected = jnp.zeros((N, D), dtype=jnp.float32)
    for i in range(L_MP):
        m, p = _layer_a(h, wct, bct, params['W_conv'][i],
                        params['b_conv'][i].reshape(1, D))
        sumS, maxS = sc_agg(m, srt, boff)
        h, selected, cp = _layer_b(sumS, maxS, m, p, deg_col, cp, selected)
    return _linear(selected, params['W_post'], params['b_post'])


# speedup vs baseline: 1.1547x; 1.1547x over previous
import functools

import jax
import jax.numpy as jnp
from jax import lax
from jax.experimental import pallas as pl
from jax.experimental.pallas import tpu as pltpu
from jax.experimental.pallas import tpu_sc as plsc

N = 10000
E = 160000
D = 256
L_MP = 4

NW = 32
NB = 12288
EPW = E // NW
EPW_PAD = 5120
CW = 128
NCH = EPW_PAD // CW
SRT_LEN = E + 12288

ROWS = 1000


@functools.lru_cache(maxsize=None)
def _build_sort_kernels():
  smesh = plsc.VectorSubcoreMesh(core_axis_name="c", subcore_axis_name="s")

  @functools.partial(
      pl.kernel,
      out_type=jax.ShapeDtypeStruct((NW, NB), jnp.int32),
      mesh=smesh,
      compiler_params=pltpu.CompilerParams(needs_layout_passes=False),
      scratch_types=[
          pltpu.VMEM((EPW_PAD,), jnp.int32),
          pltpu.VMEM((NB + 16,), jnp.int32),
      ],
  )
  def _sc_hist(dst_hbm, hist_hbm, dslab, hist):
    w = lax.axis_index("c") * 16 + lax.axis_index("s")
    pltpu.sync_copy(dst_hbm.at[pl.ds(w * EPW, EPW_PAD)], dslab)
    zz = jnp.zeros((16,), jnp.int32)

    def zr(i, _):
      hist[pl.ds(i * 16, 16)] = zz
      return 0

    lax.fori_loop(0, (NB + 16) // 16, zr, 0)

    def hr(k, _):
      d16 = dslab[pl.ds(k * 16, 16)]
      gi = k * 16 + lax.iota(jnp.int32, 16)
      d16 = jnp.where(gi < EPW, d16, NB)
      cnt, last = plsc.scan_count(d16)
      plsc.addupdate_scatter(hist, [d16], cnt, mask=last)
      return 0

    lax.fori_loop(0, EPW_PAD // 16, hr, 0)
    pltpu.sync_copy(hist.at[pl.ds(0, NB)], hist_hbm.at[w])

  @functools.partial(
      pl.kernel,
      out_type=(
          jax.ShapeDtypeStruct((NW, NB), jnp.int32),
          jax.ShapeDtypeStruct((NB,), jnp.int32),
      ),
      mesh=smesh,
      compiler_params=pltpu.CompilerParams(needs_layout_passes=False),
      scratch_types=[
          pltpu.VMEM((NW, 384), jnp.int32),
          pltpu.VMEM((NW, 384), jnp.int32),
          pltpu.VMEM((384,), jnp.int32),
      ],
  )
  def _sc_prefix(hist_hbm, wbase_hbm, tot_hbm, hv, wb, tot):
    t = lax.axis_index("c") * 16 + lax.axis_index("s")
    pltpu.sync_copy(hist_hbm.at[:, pl.ds(t * 384, 384)], hv)

    def col(kc, _):
      def row(wr, acc):
        wb[wr, pl.ds(kc * 16, 16)] = acc
        return acc + hv[wr, pl.ds(kc * 16, 16)]

      acc = lax.fori_loop(0, NW, row, jnp.zeros((16,), jnp.int32))
      tot[pl.ds(kc * 16, 16)] = acc
      return 0

    lax.fori_loop(0, 384 // 16, col, 0)
    pltpu.sync_copy(wb, wbase_hbm.at[:, pl.ds(t * 384, 384)])
    pltpu.sync_copy(tot, tot_hbm.at[pl.ds(t * 384, 384)])

  @functools.partial(
      pl.kernel,
      out_type=(
          jax.ShapeDtypeStruct((SRT_LEN,), jnp.int32),
          jax.ShapeDtypeStruct((NB + 16,), jnp.int32),
      ),
      mesh=smesh,
      compiler_params=pltpu.CompilerParams(needs_layout_passes=False),
      scratch_types=[
          pltpu.VMEM((EPW_PAD,), jnp.int32),
          pltpu.VMEM((EPW_PAD,), jnp.int32),
          pltpu.VMEM((NCH, CW), jnp.int32),
          pltpu.VMEM((NB + 16,), jnp.int32),
          pltpu.VMEM((NB + 16,), jnp.int32),
      ],
  )
  def _sc_scatter(dst_hbm, src_hbm, wbase_hbm, tot_hbm, srt_hbm, boff_hbm,
                  dslab, sslab, pos, boff, nxt):
    w = lax.axis_index("c") * 16 + lax.axis_index("s")
    pltpu.sync_copy(dst_hbm.at[pl.ds(w * EPW, EPW_PAD)], dslab)
    pltpu.sync_copy(src_hbm.at[pl.ds(w * EPW, EPW_PAD)], sslab)
    pltpu.sync_copy(tot_hbm, boff.at[pl.ds(0, NB)])
    pltpu.sync_copy(wbase_hbm.at[w], nxt.at[pl.ds(0, NB)])

    def pref(k, carry):
      v = boff[pl.ds(k * 16, 16)]
      cs = plsc.cumsum(v)
      excl = cs - v + carry
      boff[pl.ds(k * 16, 16)] = excl
      nxt[pl.ds(k * 16, 16)] = nxt[pl.ds(k * 16, 16)] + excl
      return carry + jnp.sum(v)

    lax.fori_loop(0, NB // 16, pref, jnp.int32(0))
    boff[pl.ds(NB, 16)] = jnp.zeros((16,), jnp.int32) + E
    nxt[pl.ds(NB, 16)] = jnp.zeros((16,), jnp.int32) + (E + w * CW)

    def sc(k, _):
      d16 = dslab[pl.ds(k * 16, 16)]
      gi = k * 16 + lax.iota(jnp.int32, 16)
      d16 = jnp.where(gi < EPW, d16, NB)
      cnt, last = plsc.scan_count(d16)
      base = plsc.load_gather(nxt, [d16])
      pos[k // 8, pl.ds((k % 8) * 16, 16)] = base + cnt - 1
      plsc.store_scatter(nxt, [d16], base + cnt, mask=last)
      return 0

    lax.fori_loop(0, EPW_PAD // 16, sc, 0)

    def dma(g, _):
      pltpu.sync_copy(sslab.at[pl.ds(g * CW, CW)], srt_hbm.at[pos.at[g]])
      return 0

    lax.fori_loop(0, NCH, dma, 0)

    @pl.when(w == 0)
    def _():
      pltpu.sync_copy(boff, boff_hbm)

  return _sc_hist, _sc_prefix, _sc_scatter


NPR = 320
MAXE_W = 5760
GB = 64
OB = 64
NEG = -3.0e38


@functools.lru_cache(maxsize=None)
def _build_agg_kernel():
  smesh = plsc.VectorSubcoreMesh(core_axis_name="c", subcore_axis_name="s")

  @functools.partial(
      pl.kernel,
      out_type=(
          jax.ShapeDtypeStruct((32 * NPR, 256), jnp.float32),
          jax.ShapeDtypeStruct((32 * NPR, 256), jnp.float32),
      ),
      mesh=smesh,
      compiler_params=pltpu.CompilerParams(needs_layout_passes=False),
      scratch_types=[
          pltpu.VMEM((MAXE_W + 16,), jnp.int32),
          pltpu.VMEM((MAXE_W + 16,), jnp.int32),
          pltpu.VMEM((4, GB, 256), jnp.float32),
          pltpu.VMEM((OB, 256), jnp.float32),
          pltpu.VMEM((OB, 256), jnp.float32),
          pltpu.VMEM((NPR + 16,), jnp.int32),
          pltpu.SemaphoreType.DMA,
          pltpu.SemaphoreType.DMA,
          pltpu.SemaphoreType.DMA,
          pltpu.SemaphoreType.DMA,
      ],
  )
  def _sc_agg(m2_hbm, srt_hbm, boff_hbm, sum_hbm, max_hbm,
              slab, gidx, stage, outs, outm, offs, sem0, sem1, sem2, sem3):
    nr = lax.axis_index("c") * 16 + lax.axis_index("s")
    nd0 = nr * NPR
    pltpu.sync_copy(boff_hbm.at[pl.ds(nd0, NPR + 16)], offs)

    lanes = lax.iota(jnp.int32, 16)

    def lane0(v):
      return jnp.sum(jnp.where(lanes == 0, v, 0))

    def lane1(v):
      return jnp.sum(jnp.where(lanes == 1, v, 0))

    def bin_deg(cu):
      v = offs[pl.ds(cu, 16)]
      return lane1(v) - lane0(v)

    e0 = lane0(offs[pl.ds(0, 16)])
    e1 = lane0(offs[pl.ds(NPR, 16)])
    ec = e1 - e0
    e0a = (e0 // 8) * 8
    sh = e0 - e0a
    pltpu.sync_copy(srt_hbm.at[pl.ds(e0a, MAXE_W)], slab.at[pl.ds(0, MAXE_W)])

    def tr(k, _):
      v = slab[pl.ds(sh + k * 16, 16)]
      gi = k * 16 + lax.iota(jnp.int32, 16)
      gidx[pl.ds(k * 16, 16)] = jnp.where(gi < ec, v, 0)
      return 0

    lax.fori_loop(0, (MAXE_W + 16) // 16 - 1, tr, 0)

    nch = (ec + GB - 1) // GB
    sems = (sem0, sem1, sem2, sem3)

    def issue(g, b):
      pltpu.async_copy(m2_hbm.at[gidx.at[pl.ds(g * GB, GB)]],
                       stage.at[b], sems[b])

    for _pb in range(4):
      @pl.when(nch > _pb)
      def _(_pb=_pb):
        issue(_pb, _pb)

    zero16 = jnp.zeros((16,), jnp.float32)
    neg16 = jnp.zeros((16,), jnp.float32) + NEG

    def flush_body(ob):
      row = pl.multiple_of(nd0 + ob, OB)
      pltpu.sync_copy(outs, sum_hbm.at[pl.ds(row, OB)])
      pltpu.sync_copy(outm, max_hbm.at[pl.ds(row, OB)])

    def chunk_step(g, b, carry):
      def work(carry):
        pltpu.make_async_copy(m2_hbm.at[gidx.at[pl.ds(0, GB)]],
                              stage.at[b], sems[b]).wait()
        jn = jnp.minimum(GB, ec - g * GB)

        def more(st):
          return st[0] < jn

        def per_bin(st):
          j, cur, rem, obase = st[0], st[1], st[2], st[3]
          accs = st[4:20]
          accm = st[20:36]

          def adv_cond(cr):
            return cr[1] == 0

          def adv(cr):
            cu = cr[0] + 1
            return (cu, bin_deg(cu))

          cur, rem = lax.while_loop(adv_cond, adv, (cur, rem))
          take = jnp.minimum(rem, jn - j)

          def e(t, a):
            row = [stage[b, j + t, pl.ds(k * 16, 16)] for k in range(16)]
            return (tuple(a[k] + row[k] for k in range(16))
                    + tuple(jnp.maximum(a[16 + k], row[k]) for k in range(16)))

          aa = lax.fori_loop(0, take, e, tuple(accs) + tuple(accm))
          accs, accm = aa[:16], aa[16:]
          j = j + take
          rem = rem - take

          def fin(args):
            obase, accs, accm = args

            def fl_cond(ob):
              return cur >= ob + OB

            def fl(ob):
              flush_body(ob)
              return ob + OB

            obase = lax.while_loop(fl_cond, fl, obase)
            r = cur - obase
            for k in range(16):
              outs[r, pl.ds(k * 16, 16)] = accs[k]
              outm[r, pl.ds(k * 16, 16)] = accm[k]
            return (obase,
                    tuple(zero16 for _ in range(16)),
                    tuple(neg16 for _ in range(16)))

          def nofin(args):
            obase, accs, accm = args
            return (obase, tuple(accs), tuple(accm))

          obase, accs, accm = lax.cond(
              rem == 0, fin, nofin, (obase, accs, accm))
          return (j, cur, rem, obase) + tuple(accs) + tuple(accm)

        st = (jnp.int32(0),) + tuple(carry)
        st = lax.while_loop(more, per_bin, st)
        carry = st[1:]

        @pl.when(g + 4 < nch)
        def _():
          issue(g + 4, b)

        return carry

      return lax.cond(g < nch, work, lambda c: c, carry)

    init = (jnp.int32(0), bin_deg(jnp.int32(0)), jnp.int32(0))
    init = init + tuple(jnp.zeros((16,), jnp.float32) for _ in range(16))
    init = init + tuple(jnp.zeros((16,), jnp.float32) + NEG for _ in range(16))

    def outer(g4, carry):
      for _b in range(4):
        carry = chunk_step(g4 * 4 + _b, _b, carry)
      return tuple(carry)

    carry = lax.fori_loop(0, (nch + 3) // 4, outer, init)
    flush_body(carry[2])

  return _sc_agg


def _sort_edges(edge_index):
    sc_hist, sc_prefix, sc_scatter = _build_sort_kernels()
    pad = jnp.zeros((EPW_PAD,), jnp.int32)
    dst_pad = jnp.concatenate([edge_index[1], pad])
    src_pad = jnp.concatenate([edge_index[0], pad])
    hist = sc_hist(dst_pad)
    wbase, totals = sc_prefix(hist)
    srt, boff = sc_scatter(dst_pad, src_pad, wbase, totals)
    return srt, boff, totals


def _linear_relu_kernel(x_ref, w_ref, b_ref, o_ref):
    o_ref[...] = jax.nn.relu(
        jnp.dot(x_ref[...], w_ref[...], preferred_element_type=jnp.float32)
        + b_ref[...]
    )


def _linear_kernel(x_ref, w_ref, b_ref, o_ref):
    o_ref[...] = (
        jnp.dot(x_ref[...], w_ref[...], preferred_element_type=jnp.float32)
        + b_ref[...]
    )


def _linear(x, w, b, relu=False):
    n = x.shape[0]
    return pl.pallas_call(
        _linear_relu_kernel if relu else _linear_kernel,
        grid=(n // ROWS,),
        in_specs=[
            pl.BlockSpec((ROWS, x.shape[1]), lambda i: (i, 0)),
            pl.BlockSpec((x.shape[1], w.shape[1]), lambda i: (0, 0)),
            pl.BlockSpec((1, w.shape[1]), lambda i: (0, 0)),
        ],
        out_specs=pl.BlockSpec((ROWS, w.shape[1]), lambda i: (i, 0)),
        out_shape=jax.ShapeDtypeStruct((n, w.shape[1]), jnp.float32),
    )(x, w, b.reshape(1, -1))


NBA = 16 * NPR


def _layer_a_kernel(h_ref, wct_ref, bct_ref, wc_ref, bc_ref, m_ref, p_ref):
    h = h_ref[...]
    wct = wct_ref[...]
    bct = bct_ref[...]
    ts = []
    for j in range(5):
        t = jnp.sum(h * wct[j:j + 1, :], axis=1, keepdims=True) + bct[0, j]
        ts.append(jax.nn.relu(t))
    l0, l1, a0, a1, a2 = ts
    mx01 = jnp.maximum(l0, l1)
    e0 = jnp.exp(l0 - mx01)
    e1 = jnp.exp(l1 - mx01)
    d01 = e0 + e1
    a2 = a2 - 1.0
    mx3 = jnp.maximum(jnp.maximum(a0, a1), a2)
    f0 = jnp.exp(a0 - mx3)
    f1 = jnp.exp(a1 - mx3)
    f2 = jnp.exp(a2 - mx3)
    d3 = f0 + f1 + f2
    z = jnp.zeros_like(l0)
    p_ref[...] = jnp.concatenate(
        [f0 / d3, f1 / d3, f2 / d3, e0 / d01, e1 / d01, z, z, z], axis=1)
    m_ref[...] = (
        jnp.dot(h, wc_ref[...], preferred_element_type=jnp.float32)
        + bc_ref[...]
    )


def _layer_a(h, wct, bct, wc, bc):
    return pl.pallas_call(
        _layer_a_kernel,
        grid=(N // ROWS,),
        in_specs=[
            pl.BlockSpec((ROWS, D), lambda i: (i, 0)),
            pl.BlockSpec((8, D), lambda i: (0, 0)),
            pl.BlockSpec((1, 8), lambda i: (0, 0)),
            pl.BlockSpec((D, D), lambda i: (0, 0)),
            pl.BlockSpec((1, D), lambda i: (0, 0)),
        ],
        out_specs=[
            pl.BlockSpec((ROWS, D), lambda i: (i, 0)),
            pl.BlockSpec((ROWS, 8), lambda i: (i, 0)),
        ],
        out_shape=(
            jax.ShapeDtypeStruct((N, D), jnp.float32),
            jax.ShapeDtypeStruct((N, 8), jnp.float32),
        ),
    )(h, wct, bct, wc, bc)


def _layer_b_kernel(s_ref, x_ref, m_ref, p_ref, deg_ref,
                    cp_ref, sel_ref, h2_ref, sel2_ref, cp2_ref):
    s = s_ref[...]
    mx = x_ref[...]
    m = m_ref[...]
    p = p_ref[...]
    degc = deg_ref[...]
    cp = cp_ref[...]
    pos = degc > 0.0
    mean = jnp.where(pos, s / jnp.maximum(degc, 1.0), 0.0)
    mxa = jnp.where(pos, mx, 0.0)
    out = p[:, 0:1] * mean + p[:, 1:2] * mxa + p[:, 2:3] * m
    h2 = jax.nn.relu(out)
    h2_ref[...] = h2
    sel2_ref[...] = sel_ref[...] + (cp * p[:, 4:5]) * h2
    cp2_ref[...] = cp * p[:, 3:4]


def _layer_b(sumS, maxS, m, p, deg_col, cp, sel):
    return pl.pallas_call(
        _layer_b_kernel,
        grid=(N // ROWS,),
        in_specs=[
            pl.BlockSpec((ROWS, D), lambda i: (i, 0)),
            pl.BlockSpec((ROWS, D), lambda i: (i, 0)),
            pl.BlockSpec((ROWS, D), lambda i: (i, 0)),
            pl.BlockSpec((ROWS, 8), lambda i: (i, 0)),
            pl.BlockSpec((ROWS, 1), lambda i: (i, 0)),
            pl.BlockSpec((ROWS, 1), lambda i: (i, 0)),
            pl.BlockSpec((ROWS, D), lambda i: (i, 0)),
        ],
        out_specs=[
            pl.BlockSpec((ROWS, D), lambda i: (i, 0)),
            pl.BlockSpec((ROWS, D), lambda i: (i, 0)),
            pl.BlockSpec((ROWS, 1), lambda i: (i, 0)),
        ],
        out_shape=(
            jax.ShapeDtypeStruct((N, D), jnp.float32),
            jax.ShapeDtypeStruct((N, D), jnp.float32),
            jax.ShapeDtypeStruct((N, 1), jnp.float32),
        ),
    )(sumS[:N], maxS[:N], m, p, deg_col, cp, sel)


def kernel(x, edge_index, params):
    srt, boff, totals = _sort_edges(edge_index)
    sc_agg = _build_agg_kernel()
    deg_col = totals[:N].astype(jnp.float32).reshape(N, 1)

    z3 = jnp.zeros((3, D), dtype=jnp.float32)
    wct = jnp.concatenate([params['W_dc'].T, params['W_ac'].T, z3], axis=0)
    bct = jnp.concatenate(
        [params['b_dc'], params['b_ac'], jnp.zeros((3,), jnp.float32)]
    ).reshape(1, 8)

    h = _linear(x, params['W_pre'], params['b_pre'], relu=True)
    cp = jnp.ones((N, 1), dtype=jnp.float32)
    selected = jnp.zeros((N, D), dtype=jnp.float32)
    for i in range(L_MP):
        m, p = _layer_a(h, wct, bct, params['W_conv'][i],
                        params['b_conv'][i].reshape(1, D))
        sumS, maxS = sc_agg(m, srt, boff)
        h, selected, cp = _layer_b(sumS, maxS, m, p, deg_col, cp, selected)
    return _linear(selected, params['W_post'], params['b_post'])


# speedup vs baseline: 1.2138x; 1.0512x over previous
import functools

import jax
import jax.numpy as jnp
from jax import lax
from jax.experimental import pallas as pl
from jax.experimental.pallas import tpu as pltpu
from jax.experimental.pallas import tpu_sc as plsc

N = 10000
E = 160000
D = 256
L_MP = 4

NW = 32
NB = 12288
EPW = E // NW
EPW_PAD = 5120
CW = 128
NCH = EPW_PAD // CW
SRT_LEN = E + 12288

ROWS = 1000


@functools.lru_cache(maxsize=None)
def _build_sort_kernels():
  smesh = plsc.VectorSubcoreMesh(core_axis_name="c", subcore_axis_name="s")

  @functools.partial(
      pl.kernel,
      out_type=jax.ShapeDtypeStruct((NW, NB), jnp.int32),
      mesh=smesh,
      compiler_params=pltpu.CompilerParams(needs_layout_passes=False),
      scratch_types=[
          pltpu.VMEM((EPW_PAD,), jnp.int32),
          pltpu.VMEM((NB + 16,), jnp.int32),
      ],
  )
  def _sc_hist(dst_hbm, hist_hbm, dslab, hist):
    w = lax.axis_index("c") * 16 + lax.axis_index("s")
    pltpu.sync_copy(dst_hbm.at[pl.ds(w * EPW, EPW_PAD)], dslab)
    zz = jnp.zeros((16,), jnp.int32)

    def zr(i, _):
      hist[pl.ds(i * 16, 16)] = zz
      return 0

    lax.fori_loop(0, (NB + 16) // 16, zr, 0)

    def hr(k, _):
      d16 = dslab[pl.ds(k * 16, 16)]
      gi = k * 16 + lax.iota(jnp.int32, 16)
      d16 = jnp.where(gi < EPW, d16, NB)
      cnt, last = plsc.scan_count(d16)
      plsc.addupdate_scatter(hist, [d16], cnt, mask=last)
      return 0

    lax.fori_loop(0, EPW_PAD // 16, hr, 0)
    pltpu.sync_copy(hist.at[pl.ds(0, NB)], hist_hbm.at[w])

  @functools.partial(
      pl.kernel,
      out_type=(
          jax.ShapeDtypeStruct((NW, NB), jnp.int32),
          jax.ShapeDtypeStruct((NB,), jnp.int32),
      ),
      mesh=smesh,
      compiler_params=pltpu.CompilerParams(needs_layout_passes=False),
      scratch_types=[
          pltpu.VMEM((NW, 384), jnp.int32),
          pltpu.VMEM((NW, 384), jnp.int32),
          pltpu.VMEM((384,), jnp.int32),
      ],
  )
  def _sc_prefix(hist_hbm, wbase_hbm, tot_hbm, hv, wb, tot):
    t = lax.axis_index("c") * 16 + lax.axis_index("s")
    pltpu.sync_copy(hist_hbm.at[:, pl.ds(t * 384, 384)], hv)

    def col(kc, _):
      def row(wr, acc):
        wb[wr, pl.ds(kc * 16, 16)] = acc
        return acc + hv[wr, pl.ds(kc * 16, 16)]

      acc = lax.fori_loop(0, NW, row, jnp.zeros((16,), jnp.int32))
      tot[pl.ds(kc * 16, 16)] = acc
      return 0

    lax.fori_loop(0, 384 // 16, col, 0)
    pltpu.sync_copy(wb, wbase_hbm.at[:, pl.ds(t * 384, 384)])
    pltpu.sync_copy(tot, tot_hbm.at[pl.ds(t * 384, 384)])

  @functools.partial(
      pl.kernel,
      out_type=(
          jax.ShapeDtypeStruct((SRT_LEN,), jnp.int32),
          jax.ShapeDtypeStruct((NB + 16,), jnp.int32),
      ),
      mesh=smesh,
      compiler_params=pltpu.CompilerParams(needs_layout_passes=False),
      scratch_types=[
          pltpu.VMEM((EPW_PAD,), jnp.int32),
          pltpu.VMEM((EPW_PAD,), jnp.int32),
          pltpu.VMEM((NCH, CW), jnp.int32),
          pltpu.VMEM((NB + 16,), jnp.int32),
          pltpu.VMEM((NB + 16,), jnp.int32),
      ],
  )
  def _sc_scatter(dst_hbm, src_hbm, wbase_hbm, tot_hbm, srt_hbm, boff_hbm,
                  dslab, sslab, pos, boff, nxt):
    w = lax.axis_index("c") * 16 + lax.axis_index("s")
    pltpu.sync_copy(dst_hbm.at[pl.ds(w * EPW, EPW_PAD)], dslab)
    pltpu.sync_copy(src_hbm.at[pl.ds(w * EPW, EPW_PAD)], sslab)
    pltpu.sync_copy(tot_hbm, boff.at[pl.ds(0, NB)])
    pltpu.sync_copy(wbase_hbm.at[w], nxt.at[pl.ds(0, NB)])

    def pref(k, carry):
      v = boff[pl.ds(k * 16, 16)]
      cs = plsc.cumsum(v)
      excl = cs - v + carry
      boff[pl.ds(k * 16, 16)] = excl
      nxt[pl.ds(k * 16, 16)] = nxt[pl.ds(k * 16, 16)] + excl
      return carry + jnp.sum(v)

    lax.fori_loop(0, NB // 16, pref, jnp.int32(0))
    boff[pl.ds(NB, 16)] = jnp.zeros((16,), jnp.int32) + E
    nxt[pl.ds(NB, 16)] = jnp.zeros((16,), jnp.int32) + (E + w * CW)

    def sc(k, _):
      d16 = dslab[pl.ds(k * 16, 16)]
      gi = k * 16 + lax.iota(jnp.int32, 16)
      d16 = jnp.where(gi < EPW, d16, NB)
      cnt, last = plsc.scan_count(d16)
      base = plsc.load_gather(nxt, [d16])
      pos[k // 8, pl.ds((k % 8) * 16, 16)] = base + cnt - 1
      plsc.store_scatter(nxt, [d16], base + cnt, mask=last)
      return 0

    lax.fori_loop(0, EPW_PAD // 16, sc, 0)

    def dma(g, _):
      pltpu.sync_copy(sslab.at[pl.ds(g * CW, CW)], srt_hbm.at[pos.at[g]])
      return 0

    lax.fori_loop(0, NCH, dma, 0)

    @pl.when(w == 0)
    def _():
      pltpu.sync_copy(boff, boff_hbm)

  return _sc_hist, _sc_prefix, _sc_scatter


NPR = 320
MAXE_W = 5760
GB = 32
OB = 64
NEG = -3.0e38


@functools.lru_cache(maxsize=None)
def _build_agg_kernel():
  smesh = plsc.VectorSubcoreMesh(core_axis_name="c", subcore_axis_name="s")

  @functools.partial(
      pl.kernel,
      out_type=(
          jax.ShapeDtypeStruct((32 * NPR, 256), jnp.float32),
          jax.ShapeDtypeStruct((32 * NPR, 256), jnp.float32),
      ),
      mesh=smesh,
      compiler_params=pltpu.CompilerParams(needs_layout_passes=False),
      scratch_types=[
          pltpu.VMEM((MAXE_W + 16,), jnp.int32),
          pltpu.VMEM((MAXE_W + 16,), jnp.int32),
          pltpu.VMEM((8, GB, 256), jnp.float32),
          pltpu.VMEM((OB, 256), jnp.float32),
          pltpu.VMEM((OB, 256), jnp.float32),
          pltpu.VMEM((NPR + 16,), jnp.int32),
      ] + [pltpu.SemaphoreType.DMA] * 8,
  )
  def _sc_agg(m2_hbm, srt_hbm, boff_hbm, sum_hbm, max_hbm,
              slab, gidx, stage, outs, outm, offs, *sems):
    nr = lax.axis_index("c") * 16 + lax.axis_index("s")
    nd0 = nr * NPR
    pltpu.sync_copy(boff_hbm.at[pl.ds(nd0, NPR + 16)], offs)

    lanes = lax.iota(jnp.int32, 16)

    def lane0(v):
      return jnp.sum(jnp.where(lanes == 0, v, 0))

    def lane1(v):
      return jnp.sum(jnp.where(lanes == 1, v, 0))

    def bin_deg(cu):
      v = offs[pl.ds(cu, 16)]
      return lane1(v) - lane0(v)

    e0 = lane0(offs[pl.ds(0, 16)])
    e1 = lane0(offs[pl.ds(NPR, 16)])
    ec = e1 - e0
    e0a = (e0 // 8) * 8
    sh = e0 - e0a
    pltpu.sync_copy(srt_hbm.at[pl.ds(e0a, MAXE_W)], slab.at[pl.ds(0, MAXE_W)])

    def tr(k, _):
      v = slab[pl.ds(sh + k * 16, 16)]
      gi = k * 16 + lax.iota(jnp.int32, 16)
      gidx[pl.ds(k * 16, 16)] = jnp.where(gi < ec, v, 0)
      return 0

    lax.fori_loop(0, (MAXE_W + 16) // 16 - 1, tr, 0)

    nch = (ec + GB - 1) // GB

    def issue(g, b):
      pltpu.async_copy(m2_hbm.at[gidx.at[pl.ds(g * GB, GB)]],
                       stage.at[b], sems[b])

    for _pb in range(8):
      @pl.when(nch > _pb)
      def _(_pb=_pb):
        issue(_pb, _pb)

    zero16 = jnp.zeros((16,), jnp.float32)
    neg16 = jnp.zeros((16,), jnp.float32) + NEG

    def flush_body(ob):
      row = pl.multiple_of(nd0 + ob, OB)
      pltpu.sync_copy(outs, sum_hbm.at[pl.ds(row, OB)])
      pltpu.sync_copy(outm, max_hbm.at[pl.ds(row, OB)])

    def chunk_step(g, b, carry):
      def work(carry):
        pltpu.make_async_copy(m2_hbm.at[gidx.at[pl.ds(0, GB)]],
                              stage.at[b], sems[b]).wait()
        jn = jnp.minimum(GB, ec - g * GB)

        def more(st):
          return st[0] < jn

        def per_bin(st):
          j, cur, rem, obase = st[0], st[1], st[2], st[3]
          accs = st[4:20]
          accm = st[20:36]

          def adv_cond(cr):
            return cr[1] == 0

          def adv(cr):
            cu = cr[0] + 1
            return (cu, bin_deg(cu))

          cur, rem = lax.while_loop(adv_cond, adv, (cur, rem))
          take = jnp.minimum(rem, jn - j)

          def e(t, a):
            row = [stage[b, j + t, pl.ds(k * 16, 16)] for k in range(16)]
            return (tuple(a[k] + row[k] for k in range(16))
                    + tuple(jnp.maximum(a[16 + k], row[k]) for k in range(16)))

          aa = lax.fori_loop(0, take, e, tuple(accs) + tuple(accm))
          accs, accm = aa[:16], aa[16:]
          j = j + take
          rem = rem - take

          def fin(args):
            obase, accs, accm = args

            def fl_cond(ob):
              return cur >= ob + OB

            def fl(ob):
              flush_body(ob)
              return ob + OB

            obase = lax.while_loop(fl_cond, fl, obase)
            r = cur - obase
            for k in range(16):
              outs[r, pl.ds(k * 16, 16)] = accs[k]
              outm[r, pl.ds(k * 16, 16)] = accm[k]
            return (obase,
                    tuple(zero16 for _ in range(16)),
                    tuple(neg16 for _ in range(16)))

          def nofin(args):
            obase, accs, accm = args
            return (obase, tuple(accs), tuple(accm))

          obase, accs, accm = lax.cond(
              rem == 0, fin, nofin, (obase, accs, accm))
          return (j, cur, rem, obase) + tuple(accs) + tuple(accm)

        st = (jnp.int32(0),) + tuple(carry)
        st = lax.while_loop(more, per_bin, st)
        carry = st[1:]

        @pl.when(g + 8 < nch)
        def _():
          issue(g + 8, b)

        return carry

      return lax.cond(g < nch, work, lambda c: c, carry)

    init = (jnp.int32(0), bin_deg(jnp.int32(0)), jnp.int32(0))
    init = init + tuple(jnp.zeros((16,), jnp.float32) for _ in range(16))
    init = init + tuple(jnp.zeros((16,), jnp.float32) + NEG for _ in range(16))

    def outer(g8, carry):
      for _b in range(8):
        carry = chunk_step(g8 * 8 + _b, _b, carry)
      return tuple(carry)

    carry = lax.fori_loop(0, (nch + 7) // 8, outer, init)
    flush_body(carry[2])

  return _sc_agg


def _sort_edges(edge_index):
    sc_hist, sc_prefix, sc_scatter = _build_sort_kernels()
    pad = jnp.zeros((EPW_PAD,), jnp.int32)
    dst_pad = jnp.concatenate([edge_index[1], pad])
    src_pad = jnp.concatenate([edge_index[0], pad])
    hist = sc_hist(dst_pad)
    wbase, totals = sc_prefix(hist)
    srt, boff = sc_scatter(dst_pad, src_pad, wbase, totals)
    return srt, boff, totals


def _linear_relu_kernel(x_ref, w_ref, b_ref, o_ref):
    o_ref[...] = jax.nn.relu(
        jnp.dot(x_ref[...], w_ref[...], preferred_element_type=jnp.float32)
        + b_ref[...]
    )


def _linear_kernel(x_ref, w_ref, b_ref, o_ref):
    o_ref[...] = (
        jnp.dot(x_ref[...], w_ref[...], preferred_element_type=jnp.float32)
        + b_ref[...]
    )


def _linear(x, w, b, relu=False):
    n = x.shape[0]
    return pl.pallas_call(
        _linear_relu_kernel if relu else _linear_kernel,
        grid=(n // ROWS,),
        in_specs=[
            pl.BlockSpec((ROWS, x.shape[1]), lambda i: (i, 0)),
            pl.BlockSpec((x.shape[1], w.shape[1]), lambda i: (0, 0)),
            pl.BlockSpec((1, w.shape[1]), lambda i: (0, 0)),
        ],
        out_specs=pl.BlockSpec((ROWS, w.shape[1]), lambda i: (i, 0)),
        out_shape=jax.ShapeDtypeStruct((n, w.shape[1]), jnp.float32),
    )(x, w, b.reshape(1, -1))


NBA = 16 * NPR


def _layer_a_kernel(h_ref, wct_ref, bct_ref, wc_ref, bc_ref, m_ref, p_ref):
    h = h_ref[...]
    wct = wct_ref[...]
    bct = bct_ref[...]
    ts = []
    for j in range(5):
        t = jnp.sum(h * wct[j:j + 1, :], axis=1, keepdims=True) + bct[0, j]
        ts.append(jax.nn.relu(t))
    l0, l1, a0, a1, a2 = ts
    mx01 = jnp.maximum(l0, l1)
    e0 = jnp.exp(l0 - mx01)
    e1 = jnp.exp(l1 - mx01)
    d01 = e0 + e1
    a2 = a2 - 1.0
    mx3 = jnp.maximum(jnp.maximum(a0, a1), a2)
    f0 = jnp.exp(a0 - mx3)
    f1 = jnp.exp(a1 - mx3)
    f2 = jnp.exp(a2 - mx3)
    d3 = f0 + f1 + f2
    z = jnp.zeros_like(l0)
    p_ref[...] = jnp.concatenate(
        [f0 / d3, f1 / d3, f2 / d3, e0 / d01, e1 / d01, z, z, z], axis=1)
    m_ref[...] = (
        jnp.dot(h, wc_ref[...], preferred_element_type=jnp.float32)
        + bc_ref[...]
    )


def _layer_a(h, wct, bct, wc, bc):
    return pl.pallas_call(
        _layer_a_kernel,
        grid=(N // ROWS,),
        in_specs=[
            pl.BlockSpec((ROWS, D), lambda i: (i, 0)),
            pl.BlockSpec((8, D), lambda i: (0, 0)),
            pl.BlockSpec((1, 8), lambda i: (0, 0)),
            pl.BlockSpec((D, D), lambda i: (0, 0)),
            pl.BlockSpec((1, D), lambda i: (0, 0)),
        ],
        out_specs=[
            pl.BlockSpec((ROWS, D), lambda i: (i, 0)),
            pl.BlockSpec((ROWS, 8), lambda i: (i, 0)),
        ],
        out_shape=(
            jax.ShapeDtypeStruct((N, D), jnp.float32),
            jax.ShapeDtypeStruct((N, 8), jnp.float32),
        ),
    )(h, wct, bct, wc, bc)


def _layer_b_kernel(s_ref, x_ref, m_ref, p_ref, deg_ref,
                    cp_ref, sel_ref, h2_ref, sel2_ref, cp2_ref):
    s = s_ref[...]
    mx = x_ref[...]
    m = m_ref[...]
    p = p_ref[...]
    degc = deg_ref[...]
    cp = cp_ref[...]
    pos = degc > 0.0
    mean = jnp.where(pos, s / jnp.maximum(degc, 1.0), 0.0)
    mxa = jnp.where(pos, mx, 0.0)
    out = p[:, 0:1] * mean + p[:, 1:2] * mxa + p[:, 2:3] * m
    h2 = jax.nn.relu(out)
    h2_ref[...] = h2
    sel2_ref[...] = sel_ref[...] + (cp * p[:, 4:5]) * h2
    cp2_ref[...] = cp * p[:, 3:4]


def _layer_b(sumS, maxS, m, p, deg_col, cp, sel):
    return pl.pallas_call(
        _layer_b_kernel,
        grid=(N // ROWS,),
        in_specs=[
            pl.BlockSpec((ROWS, D), lambda i: (i, 0)),
            pl.BlockSpec((ROWS, D), lambda i: (i, 0)),
            pl.BlockSpec((ROWS, D), lambda i: (i, 0)),
            pl.BlockSpec((ROWS, 8), lambda i: (i, 0)),
            pl.BlockSpec((ROWS, 1), lambda i: (i, 0)),
            pl.BlockSpec((ROWS, 1), lambda i: (i, 0)),
            pl.BlockSpec((ROWS, D), lambda i: (i, 0)),
        ],
        out_specs=[
            pl.BlockSpec((ROWS, D), lambda i: (i, 0)),
            pl.BlockSpec((ROWS, D), lambda i: (i, 0)),
            pl.BlockSpec((ROWS, 1), lambda i: (i, 0)),
        ],
        out_shape=(
            jax.ShapeDtypeStruct((N, D), jnp.float32),
            jax.ShapeDtypeStruct((N, D), jnp.float32),
            jax.ShapeDtypeStruct((N, 1), jnp.float32),
        ),
    )(sumS[:N], maxS[:N], m, p, deg_col, cp, sel)


def kernel(x, edge_index, params):
    srt, boff, totals = _sort_edges(edge_index)
    sc_agg = _build_agg_kernel()
    deg_col = totals[:N].astype(jnp.float32).reshape(N, 1)

    z3 = jnp.zeros((3, D), dtype=jnp.float32)
    wct = jnp.concatenate([params['W_dc'].T, params['W_ac'].T, z3], axis=0)
    bct = jnp.concatenate(
        [params['b_dc'], params['b_ac'], jnp.zeros((3,), jnp.float32)]
    ).reshape(1, 8)

    h = _linear(x, params['W_pre'], params['b_pre'], relu=True)
    cp = jnp.ones((N, 1), dtype=jnp.float32)
    selected = jnp.zeros((N, D), dtype=jnp.float32)
    for i in range(L_MP):
        m, p = _layer_a(h, wct, bct, params['W_conv'][i],
                        params['b_conv'][i].reshape(1, D))
        sumS, maxS = sc_agg(m, srt, boff)
        h, selected, cp = _layer_b(sumS, maxS, m, p, deg_col, cp, selected)
    return _linear(selected, params['W_post'], params['b_post'])
